# baseline (speedup 1.0000x reference)
import functools

import jax
import jax.numpy as jnp
import numpy as np
from jax import lax
from jax.experimental import pallas as pl
from jax.experimental.pallas import tpu as pltpu
from jax.experimental.pallas import tpu_sc as plsc

N = 10000
D_IN = 128
D_HID = 32
D_OUT = 64
E = 320000

NC = 2
NS = 16
NW = NC * NS
CHUNK = 128
CPT = 80
EPAD = NW * CPT * CHUNK
NROW_PAD = 10112
RPT = NROW_PAD // NS
DEG_W = 8

_mesh = plsc.VectorSubcoreMesh(core_axis_name="c", subcore_axis_name="s")


def _deg_body(col_hbm, ones_hbm, zeros_hbm, out_hbm, idx_v, ones_v, acc):
    c = lax.axis_index("c")
    s = lax.axis_index("s")
    wid = c * NS + s
    pltpu.sync_copy(zeros_hbm, acc.at[pl.ds(s * RPT, RPT)])
    pltpu.sync_copy(ones_hbm, ones_v)
    pltpu.sync_copy(col_hbm.at[pl.ds(wid * CPT, CPT)], idx_v)
    plsc.subcore_barrier()

    def step(j, _):
        pltpu.sync_copy(ones_v, acc.at[idx_v.at[j]], add=True)
        return _

    lax.fori_loop(0, CPT, step, None)
    plsc.subcore_barrier()
    pltpu.sync_copy(acc.at[pl.ds(s * RPT, RPT)],
                    out_hbm.at[c, pl.ds(s * RPT, RPT)])


_deg_kernel = pl.kernel(
    _deg_body,
    out_type=jax.ShapeDtypeStruct((NC, NROW_PAD, DEG_W), jnp.float32),
    mesh=_mesh,
    compiler_params=pltpu.CompilerParams(use_tc_tiling_on_sc=False),
    scratch_types=[
        pltpu.VMEM((CPT, CHUNK), jnp.int32),
        pltpu.VMEM((CHUNK, DEG_W), jnp.float32),
        pltpu.VMEM_SHARED((NROW_PAD, DEG_W), jnp.float32),
    ],
)


def _make_prop(D, nbuf, IB):
    def body(tbl_hbm, zeros_hbm, row_hbm, col_hbm, out_hbm, idxr_v, idxc_v,
             *rest):
        bufs = rest[:nbuf]
        gsem = rest[nbuf:2 * nbuf]
        ssem = rest[2 * nbuf:3 * nbuf]
        acc = rest[3 * nbuf]
        c = lax.axis_index("c")
        s = lax.axis_index("s")
        wid = c * NS + s

        @pl.when(c == 0)
        def _():
            pltpu.sync_copy(tbl_hbm.at[pl.ds(s * RPT, RPT)],
                            acc.at[pl.ds(s * RPT, RPT)])

        @pl.when(c == 1)
        def _():
            pltpu.sync_copy(zeros_hbm.at[pl.ds(s * RPT, RPT)],
                            acc.at[pl.ds(s * RPT, RPT)])
        plsc.subcore_barrier()

        def blk(kb, carry):
            base = wid * CPT + kb * IB
            pltpu.sync_copy(row_hbm.at[pl.ds(base, IB)], idxr_v)
            pltpu.sync_copy(col_hbm.at[pl.ds(base, IB)], idxc_v)
            for b in range(nbuf):
                pltpu.async_copy(tbl_hbm.at[idxr_v.at[b]], bufs[b], gsem[b])

            def step(it, carry2):
                j0 = it * nbuf
                for b in range(nbuf):
                    j = j0 + b
                    pltpu.make_async_copy(tbl_hbm.at[idxr_v.at[j]],
                                          bufs[b], gsem[b]).wait()
                    pltpu.async_copy(bufs[b], acc.at[idxc_v.at[j]], ssem[b],
                                     add=True)
                for b in range(nbuf):
                    j = j0 + b
                    pltpu.make_async_copy(bufs[b], acc.at[idxc_v.at[j]],
                                          ssem[b]).wait()
                    nxt = j + nbuf

                    @pl.when(nxt < IB)
                    def _():
                        pltpu.async_copy(tbl_hbm.at[idxr_v.at[nxt]],
                                         bufs[b], gsem[b])
                return carry2

            lax.fori_loop(0, IB // nbuf, step, None)
            return carry

        lax.fori_loop(0, CPT // IB, blk, None)
        plsc.subcore_barrier()
        pltpu.sync_copy(acc.at[pl.ds(s * RPT, RPT)],
                        out_hbm.at[c, pl.ds(s * RPT, RPT)])

    return pl.kernel(
        body,
        out_type=jax.ShapeDtypeStruct((NC, NROW_PAD, D), jnp.float32),
        mesh=_mesh,
        compiler_params=pltpu.CompilerParams(use_tc_tiling_on_sc=False),
        scratch_types=[
            pltpu.VMEM((IB, CHUNK), jnp.int32),
            pltpu.VMEM((IB, CHUNK), jnp.int32),
        ] + [pltpu.VMEM((CHUNK, D), jnp.float32)] * nbuf
          + [pltpu.SemaphoreType.DMA] * (2 * nbuf)
          + [pltpu.VMEM_SHARED((NROW_PAD, D), jnp.float32)],
    )


_prop64 = _make_prop(2 * D_HID, 4, 16)


def _make_prop2(nbuf, IB):
    D = 2 * D_HID

    def body(tm_hbm, tv_hbm, zeros_hbm, row_hbm, col_hbm, outm_hbm, outv_hbm,
             idxr_v, idxc_v, *rest):
        bm = rest[0:nbuf]
        bv = rest[nbuf:2 * nbuf]
        gm = rest[2 * nbuf:3 * nbuf]
        gv = rest[3 * nbuf:4 * nbuf]
        sm = rest[4 * nbuf:5 * nbuf]
        sv = rest[5 * nbuf:6 * nbuf]
        accm = rest[6 * nbuf]
        accv = rest[6 * nbuf + 1]
        c = lax.axis_index("c")
        s = lax.axis_index("s")
        wid = c * NS + s

        @pl.when(c == 0)
        def _():
            pltpu.sync_copy(tm_hbm.at[pl.ds(s * RPT, RPT)],
                            accm.at[pl.ds(s * RPT, RPT)])
            pltpu.sync_copy(tv_hbm.at[pl.ds(s * RPT, RPT)],
                            accv.at[pl.ds(s * RPT, RPT)])

        @pl.when(c == 1)
        def _():
            pltpu.sync_copy(zeros_hbm.at[pl.ds(s * RPT, RPT)],
                            accm.at[pl.ds(s * RPT, RPT)])
            pltpu.sync_copy(zeros_hbm.at[pl.ds(s * RPT, RPT)],
                            accv.at[pl.ds(s * RPT, RPT)])
        plsc.subcore_barrier()

        def blk(kb, carry):
            base = wid * CPT + kb * IB
            pltpu.sync_copy(row_hbm.at[pl.ds(base, IB)], idxr_v)
            pltpu.sync_copy(col_hbm.at[pl.ds(base, IB)], idxc_v)
            for b in range(nbuf):
                pltpu.async_copy(tm_hbm.at[idxr_v.at[b]], bm[b], gm[b])
                pltpu.async_copy(tv_hbm.at[idxr_v.at[b]], bv[b], gv[b])

            def step(it, carry2):
                j0 = it * nbuf
                for b in range(nbuf):
                    j = j0 + b
                    pltpu.make_async_copy(tm_hbm.at[idxr_v.at[j]],
                                          bm[b], gm[b]).wait()
                    pltpu.async_copy(bm[b], accm.at[idxc_v.at[j]], sm[b],
                                     add=True)
                    pltpu.make_async_copy(tv_hbm.at[idxr_v.at[j]],
                                          bv[b], gv[b]).wait()
                    pltpu.async_copy(bv[b], accv.at[idxc_v.at[j]], sv[b],
                                     add=True)
                for b in range(nbuf):
                    j = j0 + b
                    nxt = j + nbuf
                    pltpu.make_async_copy(bm[b], accm.at[idxc_v.at[j]],
                                          sm[b]).wait()

                    @pl.when(nxt < IB)
                    def _():
                        pltpu.async_copy(tm_hbm.at[idxr_v.at[nxt]],
                                         bm[b], gm[b])
                    pltpu.make_async_copy(bv[b], accv.at[idxc_v.at[j]],
                                          sv[b]).wait()

                    @pl.when(nxt < IB)
                    def _():
                        pltpu.async_copy(tv_hbm.at[idxr_v.at[nxt]],
                                         bv[b], gv[b])
                return carry2

            lax.fori_loop(0, IB // nbuf, step, None)
            return carry

        lax.fori_loop(0, CPT // IB, blk, None)
        plsc.subcore_barrier()
        pltpu.sync_copy(accm.at[pl.ds(s * RPT, RPT)],
                        outm_hbm.at[c, pl.ds(s * RPT, RPT)])
        pltpu.sync_copy(accv.at[pl.ds(s * RPT, RPT)],
                        outv_hbm.at[c, pl.ds(s * RPT, RPT)])

    return pl.kernel(
        body,
        out_type=[jax.ShapeDtypeStruct((NC, NROW_PAD, D), jnp.float32),
                  jax.ShapeDtypeStruct((NC, NROW_PAD, D), jnp.float32)],
        mesh=_mesh,
        compiler_params=pltpu.CompilerParams(use_tc_tiling_on_sc=False),
        scratch_types=[
            pltpu.VMEM((IB, CHUNK), jnp.int32),
            pltpu.VMEM((IB, CHUNK), jnp.int32),
        ] + [pltpu.VMEM((CHUNK, D), jnp.float32)] * (2 * nbuf)
          + [pltpu.SemaphoreType.DMA] * (4 * nbuf)
          + [pltpu.VMEM_SHARED((NROW_PAD, D), jnp.float32)] * 2,
    )


_prop2 = _make_prop2(2, 16)


_EB = 160
_ECH = EPAD // CHUNK


def _edges_body(e_ref, rowp_ref, colp_ref):
    i = pl.program_id(0)
    e = e_ref[...]
    r = i * _EB + lax.broadcasted_iota(jnp.int32, (_EB, CHUNK), 0)
    cc = lax.broadcasted_iota(jnp.int32, (_EB, CHUNK), 1)
    flat = r * CHUNK + cc
    valid = flat < E
    padv = N + ((flat - E) % (NROW_PAD - N))
    rowp_ref[...] = jnp.where(valid, e[0], padv)
    colp_ref[...] = jnp.where(valid, e[1], padv)


_edges = pl.pallas_call(
    _edges_body,
    grid=(_ECH // _EB,),
    in_specs=[pl.BlockSpec((2, _EB, CHUNK), lambda i: (0, i, 0))],
    out_specs=[pl.BlockSpec((_EB, CHUNK), lambda i: (i, 0)),
               pl.BlockSpec((_EB, CHUNK), lambda i: (i, 0))],
    out_shape=[jax.ShapeDtypeStruct((_ECH, CHUNK), jnp.int32),
               jax.ShapeDtypeStruct((_ECH, CHUNK), jnp.int32)],
)


_BLK = 5056
_GRID_N = NROW_PAD // _BLK


def _dis_from_deg(deg_ref):
    d = deg_ref[...]
    deg = d[0, :, 0:1] + d[1, :, 0:1] + 1.0
    dis2 = 1.0 / deg
    return lax.rsqrt(deg), dis2


def _rowmask(i):
    r = i * _BLK + lax.broadcasted_iota(jnp.int32, (_BLK, 1), 0)
    return r < N


def _dense1_body(x_ref, w1m_ref, b1m_ref, w1v_ref, b1v_ref, deg_ref, out_ref):
    i = pl.program_id(0)
    dis, dis2 = _dis_from_deg(deg_ref)
    xb = x_ref[...]
    m = jax.nn.relu(jnp.dot(xb, w1m_ref[...],
                            preferred_element_type=jnp.float32) + b1m_ref[...])
    v = jax.nn.relu(jnp.dot(xb, w1v_ref[...],
                            preferred_element_type=jnp.float32) + b1v_ref[...])
    att = jnp.exp(-v)
    ma = m * att
    va = v * att * att
    t = jnp.concatenate([ma * dis, va * dis2], axis=1)
    out_ref[...] = jnp.where(_rowmask(i), t, 0.0)


def _dense2_body(acc_ref, w2m_ref, b2m_ref, w2v_ref, b2v_ref, deg_ref,
                 outm_ref, outv_ref):
    i = pl.program_id(0)
    dis, dis2 = _dis_from_deg(deg_ref)
    a = acc_ref[...]
    asum = a[0] + a[1]
    hm = jax.nn.relu(asum[:, :D_HID] * dis)
    hv = jax.nn.relu(asum[:, D_HID:] * dis2)
    m = jax.nn.relu(jnp.dot(hm, w2m_ref[...],
                            preferred_element_type=jnp.float32) + b2m_ref[...])
    v = jax.nn.relu(jnp.dot(hv, w2v_ref[...],
                            preferred_element_type=jnp.float32) + b2v_ref[...])
    att = jnp.exp(-v)
    ma = m * att
    va = v * att * att
    msk = _rowmask(i)
    outm_ref[...] = jnp.where(msk, ma * dis, 0.0)
    outv_ref[...] = jnp.where(msk, va * dis2, 0.0)


def _final_body(accm_ref, accv_ref, deg_ref, eps_ref, out_ref):
    dis, dis2 = _dis_from_deg(deg_ref)
    am = accm_ref[...]
    av = accv_ref[...]
    mean = (am[0] + am[1]) * dis
    var = (av[0] + av[1]) * dis2
    out_ref[...] = eps_ref[...] * jnp.sqrt(var + 1e-8) + mean


_full = lambda shape: pl.BlockSpec(shape, lambda i: (0,) * len(shape))
_row_block = lambda shape2: pl.BlockSpec((_BLK,) + tuple(shape2[1:]),
                                         lambda i: (i,) + (0,) * (len(shape2) - 1))
_deg_spec = pl.BlockSpec((NC, _BLK, DEG_W), lambda i: (0, i, 0))


_dense1 = pl.pallas_call(
    _dense1_body,
    grid=(_GRID_N,),
    in_specs=[
        _row_block((N, D_IN)),
        _full((D_IN, D_HID)), _full((1, D_HID)),
        _full((D_IN, D_HID)), _full((1, D_HID)),
        _deg_spec,
    ],
    out_specs=_row_block((NROW_PAD, 2 * D_HID)),
    out_shape=jax.ShapeDtypeStruct((NROW_PAD, 2 * D_HID), jnp.float32),
)

_dense2 = pl.pallas_call(
    _dense2_body,
    grid=(_GRID_N,),
    in_specs=[
        pl.BlockSpec((NC, _BLK, 2 * D_HID), lambda i: (0, i, 0)),
        _full((D_HID, D_OUT)), _full((1, D_OUT)),
        _full((D_HID, D_OUT)), _full((1, D_OUT)),
        _deg_spec,
    ],
    out_specs=[_row_block((NROW_PAD, D_OUT)), _row_block((NROW_PAD, D_OUT))],
    out_shape=[jax.ShapeDtypeStruct((NROW_PAD, D_OUT), jnp.float32),
               jax.ShapeDtypeStruct((NROW_PAD, D_OUT), jnp.float32)],
)

_final = pl.pallas_call(
    _final_body,
    grid=(_GRID_N,),
    in_specs=[
        pl.BlockSpec((NC, _BLK, D_OUT), lambda i: (0, i, 0)),
        pl.BlockSpec((NC, _BLK, D_OUT), lambda i: (0, i, 0)),
        _deg_spec,
        _row_block((N, D_OUT)),
    ],
    out_specs=_row_block((N, D_OUT)),
    out_shape=jax.ShapeDtypeStruct((N, D_OUT), jnp.float32),
)


_EPS_CACHE = []


def _eps():
    if not _EPS_CACHE:
        try:
            with jax.ensure_compile_time_eval():
                e = jax.random.normal(jax.random.key(42), (N, D_OUT),
                                      dtype=jnp.float32)
            _EPS_CACHE.append(np.asarray(e))
        except Exception:
            return jax.random.normal(jax.random.key(42), (N, D_OUT),
                                     dtype=jnp.float32)
    return jnp.asarray(_EPS_CACHE[0])


@jax.jit
def kernel(x, edge_index, W1m, b1m, W1v, b1v, W2m, b2m, W2v, b2v):
    ei = edge_index.astype(jnp.int32).reshape(2, E // CHUNK, CHUNK)
    rowp, colp = _edges(ei)

    ones_rows = jnp.ones((CHUNK, DEG_W), jnp.float32)
    zero_rows = jnp.zeros((RPT, DEG_W), jnp.float32)
    deg_p = _deg_kernel(colp, ones_rows, zero_rows)

    b1m2 = b1m.reshape(1, D_HID)
    b1v2 = b1v.reshape(1, D_HID)
    b2m2 = b2m.reshape(1, D_OUT)
    b2v2 = b2v.reshape(1, D_OUT)

    z64 = jnp.zeros((NROW_PAD, 2 * D_HID), jnp.float32)
    t1 = _dense1(x, W1m, b1m2, W1v, b1v2, deg_p)
    acc1 = _prop64(t1, z64, rowp, colp)

    tm, tv = _dense2(acc1, W2m, b2m2, W2v, b2v2, deg_p)
    acc2m, acc2v = _prop2(tm, tv, z64, rowp, colp)

    return _final(acc2m, acc2v, deg_p, _eps())


# speedup vs baseline: 1.0035x; 1.0035x over previous
import jax
import jax.numpy as jnp
import numpy as np
from jax import lax
from jax.experimental import pallas as pl
from jax.experimental.pallas import tpu as pltpu
from jax.experimental.pallas import tpu_sc as plsc

N = 10000
D_IN = 128
D_HID = 32
D_OUT = 64
E = 320000

NC = 2
NS = 16
NW = NC * NS
CHUNK = 128
CPT = 80
EPAD = NW * CPT * CHUNK
NROW_PAD = 10112
RPT = NROW_PAD // NS
DEG_W = 8

_mesh = plsc.VectorSubcoreMesh(core_axis_name="c", subcore_axis_name="s")


def _deg_body(col_hbm, ones_hbm, zeros_hbm, out_hbm, idx_v, ones_v, acc):
    c = lax.axis_index("c")
    s = lax.axis_index("s")
    wid = c * NS + s
    pltpu.sync_copy(zeros_hbm, acc.at[pl.ds(s * RPT, RPT)])
    pltpu.sync_copy(ones_hbm, ones_v)
    pltpu.sync_copy(col_hbm.at[pl.ds(wid * CPT, CPT)], idx_v)
    plsc.subcore_barrier()

    def step(j, _):
        pltpu.sync_copy(ones_v, acc.at[idx_v.at[j]], add=True)
        return _

    lax.fori_loop(0, CPT, step, None)
    plsc.subcore_barrier()
    pltpu.sync_copy(acc.at[pl.ds(s * RPT, RPT)],
                    out_hbm.at[c, pl.ds(s * RPT, RPT)])


_deg_kernel = pl.kernel(
    _deg_body,
    out_type=jax.ShapeDtypeStruct((NC, NROW_PAD, DEG_W), jnp.float32),
    mesh=_mesh,
    compiler_params=pltpu.CompilerParams(use_tc_tiling_on_sc=False),
    scratch_types=[
        pltpu.VMEM((CPT, CHUNK), jnp.int32),
        pltpu.VMEM((CHUNK, DEG_W), jnp.float32),
        pltpu.VMEM_SHARED((NROW_PAD, DEG_W), jnp.float32),
    ],
)


def _make_prop(D, nbuf, IB):
    def body(tbl_hbm, zeros_hbm, row_hbm, col_hbm, out_hbm, idxr_v, idxc_v,
             *rest):
        bufs = rest[:nbuf]
        gsem = rest[nbuf:2 * nbuf]
        ssem = rest[2 * nbuf:3 * nbuf]
        acc = rest[3 * nbuf]
        c = lax.axis_index("c")
        s = lax.axis_index("s")
        wid = c * NS + s

        @pl.when(c == 0)
        def _():
            pltpu.sync_copy(tbl_hbm.at[pl.ds(s * RPT, RPT)],
                            acc.at[pl.ds(s * RPT, RPT)])

        @pl.when(c == 1)
        def _():
            pltpu.sync_copy(zeros_hbm.at[pl.ds(s * RPT, RPT)],
                            acc.at[pl.ds(s * RPT, RPT)])
        plsc.subcore_barrier()

        def blk(kb, carry):
            base = wid * CPT + kb * IB
            pltpu.sync_copy(row_hbm.at[pl.ds(base, IB)], idxr_v)
            pltpu.sync_copy(col_hbm.at[pl.ds(base, IB)], idxc_v)
            for b in range(nbuf):
                pltpu.async_copy(tbl_hbm.at[idxr_v.at[b]], bufs[b], gsem[b])

            def step(it, carry2):
                j0 = it * nbuf
                for b in range(nbuf):
                    j = j0 + b
                    pltpu.make_async_copy(tbl_hbm.at[idxr_v.at[j]],
                                          bufs[b], gsem[b]).wait()
                    pltpu.async_copy(bufs[b], acc.at[idxc_v.at[j]], ssem[b],
                                     add=True)
                for b in range(nbuf):
                    j = j0 + b
                    pltpu.make_async_copy(bufs[b], acc.at[idxc_v.at[j]],
                                          ssem[b]).wait()
                    nxt = j + nbuf

                    @pl.when(nxt < IB)
                    def _():
                        pltpu.async_copy(tbl_hbm.at[idxr_v.at[nxt]],
                                         bufs[b], gsem[b])
                return carry2

            lax.fori_loop(0, IB // nbuf, step, None)
            return carry

        lax.fori_loop(0, CPT // IB, blk, None)
        plsc.subcore_barrier()
        pltpu.sync_copy(acc.at[pl.ds(s * RPT, RPT)],
                        out_hbm.at[c, pl.ds(s * RPT, RPT)])

    return pl.kernel(
        body,
        out_type=jax.ShapeDtypeStruct((NC, NROW_PAD, D), jnp.float32),
        mesh=_mesh,
        compiler_params=pltpu.CompilerParams(use_tc_tiling_on_sc=False),
        scratch_types=[
            pltpu.VMEM((IB, CHUNK), jnp.int32),
            pltpu.VMEM((IB, CHUNK), jnp.int32),
        ] + [pltpu.VMEM((CHUNK, D), jnp.float32)] * nbuf
          + [pltpu.SemaphoreType.DMA] * (2 * nbuf)
          + [pltpu.VMEM_SHARED((NROW_PAD, D), jnp.float32)],
    )


_prop64 = _make_prop(2 * D_HID, 4, 16)


def _make_prop2(nbuf, IB):
    D = 2 * D_HID

    def body(tm_hbm, tv_hbm, zeros_hbm, row_hbm, col_hbm, outm_hbm, outv_hbm,
             idxr_v, idxc_v, *rest):
        bm = rest[0:nbuf]
        bv = rest[nbuf:2 * nbuf]
        gm = rest[2 * nbuf:3 * nbuf]
        gv = rest[3 * nbuf:4 * nbuf]
        sm = rest[4 * nbuf:5 * nbuf]
        sv = rest[5 * nbuf:6 * nbuf]
        accm = rest[6 * nbuf]
        accv = rest[6 * nbuf + 1]
        c = lax.axis_index("c")
        s = lax.axis_index("s")
        wid = c * NS + s

        @pl.when(c == 0)
        def _():
            pltpu.sync_copy(tm_hbm.at[pl.ds(s * RPT, RPT)],
                            accm.at[pl.ds(s * RPT, RPT)])
            pltpu.sync_copy(tv_hbm.at[pl.ds(s * RPT, RPT)],
                            accv.at[pl.ds(s * RPT, RPT)])

        @pl.when(c == 1)
        def _():
            pltpu.sync_copy(zeros_hbm.at[pl.ds(s * RPT, RPT)],
                            accm.at[pl.ds(s * RPT, RPT)])
            pltpu.sync_copy(zeros_hbm.at[pl.ds(s * RPT, RPT)],
                            accv.at[pl.ds(s * RPT, RPT)])
        plsc.subcore_barrier()

        def blk(kb, carry):
            base = wid * CPT + kb * IB
            pltpu.sync_copy(row_hbm.at[pl.ds(base, IB)], idxr_v)
            pltpu.sync_copy(col_hbm.at[pl.ds(base, IB)], idxc_v)
            for b in range(nbuf):
                pltpu.async_copy(tm_hbm.at[idxr_v.at[b]], bm[b], gm[b])
                pltpu.async_copy(tv_hbm.at[idxr_v.at[b]], bv[b], gv[b])

            def step(it, carry2):
                j0 = it * nbuf
                for b in range(nbuf):
                    j = j0 + b
                    pltpu.make_async_copy(tm_hbm.at[idxr_v.at[j]],
                                          bm[b], gm[b]).wait()
                    pltpu.async_copy(bm[b], accm.at[idxc_v.at[j]], sm[b],
                                     add=True)
                    pltpu.make_async_copy(tv_hbm.at[idxr_v.at[j]],
                                          bv[b], gv[b]).wait()
                    pltpu.async_copy(bv[b], accv.at[idxc_v.at[j]], sv[b],
                                     add=True)
                for b in range(nbuf):
                    j = j0 + b
                    nxt = j + nbuf
                    pltpu.make_async_copy(bm[b], accm.at[idxc_v.at[j]],
                                          sm[b]).wait()

                    @pl.when(nxt < IB)
                    def _():
                        pltpu.async_copy(tm_hbm.at[idxr_v.at[nxt]],
                                         bm[b], gm[b])
                    pltpu.make_async_copy(bv[b], accv.at[idxc_v.at[j]],
                                          sv[b]).wait()

                    @pl.when(nxt < IB)
                    def _():
                        pltpu.async_copy(tv_hbm.at[idxr_v.at[nxt]],
                                         bv[b], gv[b])
                return carry2

            lax.fori_loop(0, IB // nbuf, step, None)
            return carry

        lax.fori_loop(0, CPT // IB, blk, None)
        plsc.subcore_barrier()
        pltpu.sync_copy(accm.at[pl.ds(s * RPT, RPT)],
                        outm_hbm.at[c, pl.ds(s * RPT, RPT)])
        pltpu.sync_copy(accv.at[pl.ds(s * RPT, RPT)],
                        outv_hbm.at[c, pl.ds(s * RPT, RPT)])

    return pl.kernel(
        body,
        out_type=[jax.ShapeDtypeStruct((NC, NROW_PAD, D), jnp.float32),
                  jax.ShapeDtypeStruct((NC, NROW_PAD, D), jnp.float32)],
        mesh=_mesh,
        compiler_params=pltpu.CompilerParams(use_tc_tiling_on_sc=False),
        scratch_types=[
            pltpu.VMEM((IB, CHUNK), jnp.int32),
            pltpu.VMEM((IB, CHUNK), jnp.int32),
        ] + [pltpu.VMEM((CHUNK, D), jnp.float32)] * (2 * nbuf)
          + [pltpu.SemaphoreType.DMA] * (4 * nbuf)
          + [pltpu.VMEM_SHARED((NROW_PAD, D), jnp.float32)] * 2,
    )


_prop2 = _make_prop2(2, 16)


_EB = 160
_ECH = EPAD // CHUNK


def _edges_body(e_ref, rowp_ref, colp_ref):
    i = pl.program_id(0)
    e = e_ref[...]
    r = i * _EB + lax.broadcasted_iota(jnp.int32, (_EB, CHUNK), 0)
    cc = lax.broadcasted_iota(jnp.int32, (_EB, CHUNK), 1)
    flat = r * CHUNK + cc
    valid = flat < E
    padv = N + ((flat - E) % (NROW_PAD - N))
    rowp_ref[...] = jnp.where(valid, e[0], padv)
    colp_ref[...] = jnp.where(valid, e[1], padv)


_edges = pl.pallas_call(
    _edges_body,
    grid=(_ECH // _EB,),
    in_specs=[pl.BlockSpec((2, _EB, CHUNK), lambda i: (0, i, 0))],
    out_specs=[pl.BlockSpec((_EB, CHUNK), lambda i: (i, 0)),
               pl.BlockSpec((_EB, CHUNK), lambda i: (i, 0))],
    out_shape=[jax.ShapeDtypeStruct((_ECH, CHUNK), jnp.int32),
               jax.ShapeDtypeStruct((_ECH, CHUNK), jnp.int32)],
)


_BLK = 2528
_GRID_N = NROW_PAD // _BLK


def _dis_from_deg(deg_ref):
    d = deg_ref[...]
    deg = d[0, :, 0:1] + d[1, :, 0:1] + 1.0
    dis2 = 1.0 / deg
    return lax.rsqrt(deg), dis2


def _rowmask(i):
    r = i * _BLK + lax.broadcasted_iota(jnp.int32, (_BLK, 1), 0)
    return r < N


def _dense1_body(x_ref, w1m_ref, b1m_ref, w1v_ref, b1v_ref, deg_ref, out_ref):
    i = pl.program_id(0)
    dis, dis2 = _dis_from_deg(deg_ref)
    xb = x_ref[...]
    m = jax.nn.relu(jnp.dot(xb, w1m_ref[...],
                            preferred_element_type=jnp.float32) + b1m_ref[...])
    v = jax.nn.relu(jnp.dot(xb, w1v_ref[...],
                            preferred_element_type=jnp.float32) + b1v_ref[...])
    att = jnp.exp(-v)
    ma = m * att
    va = v * att * att
    t = jnp.concatenate([ma * dis, va * dis2], axis=1)
    out_ref[...] = jnp.where(_rowmask(i), t, 0.0)


def _dense2_body(acc_ref, w2m_ref, b2m_ref, w2v_ref, b2v_ref, deg_ref,
                 outm_ref, outv_ref):
    i = pl.program_id(0)
    dis, dis2 = _dis_from_deg(deg_ref)
    a = acc_ref[...]
    asum = a[0] + a[1]
    hm = jax.nn.relu(asum[:, :D_HID] * dis)
    hv = jax.nn.relu(asum[:, D_HID:] * dis2)
    m = jax.nn.relu(jnp.dot(hm, w2m_ref[...],
                            preferred_element_type=jnp.float32) + b2m_ref[...])
    v = jax.nn.relu(jnp.dot(hv, w2v_ref[...],
                            preferred_element_type=jnp.float32) + b2v_ref[...])
    att = jnp.exp(-v)
    ma = m * att
    va = v * att * att
    msk = _rowmask(i)
    outm_ref[...] = jnp.where(msk, ma * dis, 0.0)
    outv_ref[...] = jnp.where(msk, va * dis2, 0.0)


def _final_body(accm_ref, accv_ref, deg_ref, eps_ref, out_ref):
    dis, dis2 = _dis_from_deg(deg_ref)
    am = accm_ref[...]
    av = accv_ref[...]
    mean = (am[0] + am[1]) * dis
    var = (av[0] + av[1]) * dis2
    out_ref[...] = eps_ref[...] * jnp.sqrt(var + 1e-8) + mean


_full = lambda shape: pl.BlockSpec(shape, lambda i: (0,) * len(shape))
_row_block = lambda shape2: pl.BlockSpec((_BLK,) + tuple(shape2[1:]),
                                         lambda i: (i,) + (0,) * (len(shape2) - 1))
_deg_spec = pl.BlockSpec((NC, _BLK, DEG_W), lambda i: (0, i, 0))


_dense1 = pl.pallas_call(
    _dense1_body,
    grid=(_GRID_N,),
    in_specs=[
        _row_block((N, D_IN)),
        _full((D_IN, D_HID)), _full((1, D_HID)),
        _full((D_IN, D_HID)), _full((1, D_HID)),
        _deg_spec,
    ],
    out_specs=_row_block((NROW_PAD, 2 * D_HID)),
    out_shape=jax.ShapeDtypeStruct((NROW_PAD, 2 * D_HID), jnp.float32),
)

_dense2 = pl.pallas_call(
    _dense2_body,
    grid=(_GRID_N,),
    in_specs=[
        pl.BlockSpec((NC, _BLK, 2 * D_HID), lambda i: (0, i, 0)),
        _full((D_HID, D_OUT)), _full((1, D_OUT)),
        _full((D_HID, D_OUT)), _full((1, D_OUT)),
        _deg_spec,
    ],
    out_specs=[_row_block((NROW_PAD, D_OUT)), _row_block((NROW_PAD, D_OUT))],
    out_shape=[jax.ShapeDtypeStruct((NROW_PAD, D_OUT), jnp.float32),
               jax.ShapeDtypeStruct((NROW_PAD, D_OUT), jnp.float32)],
)

_final = pl.pallas_call(
    _final_body,
    grid=(_GRID_N,),
    in_specs=[
        pl.BlockSpec((NC, _BLK, D_OUT), lambda i: (0, i, 0)),
        pl.BlockSpec((NC, _BLK, D_OUT), lambda i: (0, i, 0)),
        _deg_spec,
        _row_block((N, D_OUT)),
    ],
    out_specs=_row_block((N, D_OUT)),
    out_shape=jax.ShapeDtypeStruct((N, D_OUT), jnp.float32),
)


_EPS_CACHE = []


def _eps():
    if not _EPS_CACHE:
        try:
            with jax.ensure_compile_time_eval():
                e = jax.random.normal(jax.random.key(42), (N, D_OUT),
                                      dtype=jnp.float32)
            _EPS_CACHE.append(np.asarray(e))
        except Exception:
            return jax.random.normal(jax.random.key(42), (N, D_OUT),
                                     dtype=jnp.float32)
    return jnp.asarray(_EPS_CACHE[0])


@jax.jit
def kernel(x, edge_index, W1m, b1m, W1v, b1v, W2m, b2m, W2v, b2v):
    ei = edge_index.astype(jnp.int32).reshape(2, E // CHUNK, CHUNK)
    rowp, colp = _edges(ei)

    ones_rows = jnp.ones((CHUNK, DEG_W), jnp.float32)
    zero_rows = jnp.zeros((RPT, DEG_W), jnp.float32)
    deg_p = _deg_kernel(colp, ones_rows, zero_rows)

    b1m2 = b1m.reshape(1, D_HID)
    b1v2 = b1v.reshape(1, D_HID)
    b2m2 = b2m.reshape(1, D_OUT)
    b2v2 = b2v.reshape(1, D_OUT)

    z64 = jnp.zeros((NROW_PAD, 2 * D_HID), jnp.float32)
    t1 = _dense1(x, W1m, b1m2, W1v, b1v2, deg_p)
    acc1 = _prop64(t1, z64, rowp, colp)

    tm, tv = _dense2(acc1, W2m, b2m2, W2v, b2v2, deg_p)
    acc2m, acc2v = _prop2(tm, tv, z64, rowp, colp)

    return _final(acc2m, acc2v, deg_p, _eps())
